# Initial kernel scaffold; baseline (speedup 1.0000x reference)
import jax
import jax.numpy as jnp
from jax.experimental import pallas as pl


def kernel(z, edge_index, W0, b0, W1, b1, W2, b2, W3, b3, Wr1, br1, Wr2, br2, Wt1, bt1, Wt2, bt2):
    raise NotImplementedError("write your pallas kernel here")


# speedup vs baseline: 5.6639x; 5.6639x over previous
import functools

import jax
import jax.numpy as jnp
from jax import lax
from jax.experimental import pallas as pl
from jax.experimental.pallas import tpu as pltpu
from jax.experimental.pallas import tpu_sc as plsc

NN = 10000
NP = 10240
EE = 320000
DD = 128
HH = 128
CW = 256

NC = 2
NS = 16
NW = NC * NS
EPT = EE // NW
EK = 80
RPT = NP // NS
RCH = 160

_MESH = plsc.VectorSubcoreMesh(core_axis_name="c", subcore_axis_name="s")


def _make_seg_gather_add():
  @functools.partial(
      pl.kernel,
      out_type=jax.ShapeDtypeStruct((NC * NP, HH), jnp.float32),
      mesh=_MESH,
      scratch_types=[
          pltpu.VMEM((EK,), jnp.int32),
          pltpu.VMEM((EK,), jnp.int32),
          pltpu.VMEM((EK, HH), jnp.float32),
          pltpu.VMEM((RCH, HH), jnp.float32),
          pltpu.VMEM_SHARED((NP, HH), jnp.float32),
          pltpu.SemaphoreType.DMA,
      ],
  )
  def seg(table, src, dst, out, sidx, didx, rows, stage, acc, sem):
    c = lax.axis_index("c")
    s = lax.axis_index("s")
    wid = s * NC + c

    rbase = s * RPT

    def init_body(i, carry):
      off = rbase + i * RCH
      pltpu.sync_copy(table.at[pl.ds(off, RCH)], stage)
      pltpu.sync_copy(stage, acc.at[pl.ds(off, RCH)])
      return carry

    lax.fori_loop(0, RPT // RCH, init_body, 0)
    plsc.subcore_barrier()

    ebase = wid * EPT

    def edge_body(j, carry):
      off = ebase + j * EK
      pltpu.sync_copy(src.at[pl.ds(off, EK)], sidx)
      pltpu.sync_copy(dst.at[pl.ds(off, EK)], didx)
      pltpu.async_copy(table.at[sidx], rows, sem).wait()
      pltpu.sync_copy(rows, acc.at[didx], add=True)
      return carry

    lax.fori_loop(0, EPT // EK, edge_body, 0)
    plsc.subcore_barrier()

    obase = c * NP + rbase

    def out_body(i, carry):
      pltpu.sync_copy(acc.at[pl.ds(rbase + i * RCH, RCH)], stage)
      pltpu.sync_copy(stage, out.at[pl.ds(obase + i * RCH, RCH)])
      return carry

    lax.fori_loop(0, RPT // RCH, out_body, 0)

  return seg


_seg_gather_add = _make_seg_gather_add()


def _make_deg_count():
  @functools.partial(
      pl.kernel,
      out_type=jax.ShapeDtypeStruct((NC * NP, HH), jnp.float32),
      mesh=_MESH,
      scratch_types=[
          pltpu.VMEM((EK,), jnp.int32),
          pltpu.VMEM((EK, HH), jnp.float32),
          pltpu.VMEM((RCH, HH), jnp.float32),
          pltpu.VMEM_SHARED((NP, HH), jnp.float32),
          pltpu.SemaphoreType.DMA,
      ],
  )
  def degk(zeros_tab, ones_blk, dst, out, didx, rows, stage, acc, sem):
    c = lax.axis_index("c")
    s = lax.axis_index("s")
    wid = s * NC + c

    rbase = s * RPT

    def init_body(i, carry):
      off = rbase + i * RCH
      pltpu.sync_copy(zeros_tab.at[pl.ds(off, RCH)], stage)
      pltpu.sync_copy(stage, acc.at[pl.ds(off, RCH)])
      return carry

    lax.fori_loop(0, RPT // RCH, init_body, 0)
    pltpu.sync_copy(ones_blk, rows)
    plsc.subcore_barrier()

    ebase = wid * EPT

    def edge_body(j, carry):
      off = ebase + j * EK
      pltpu.sync_copy(dst.at[pl.ds(off, EK)], didx)
      pltpu.sync_copy(rows, acc.at[didx], add=True)
      return carry

    lax.fori_loop(0, EPT // EK, edge_body, 0)
    plsc.subcore_barrier()

    obase = c * NP + rbase

    def out_body(i, carry):
      pltpu.sync_copy(acc.at[pl.ds(rbase + i * RCH, RCH)], stage)
      pltpu.sync_copy(stage, out.at[pl.ds(obase + i * RCH, RCH)])
      return carry

    lax.fori_loop(0, RPT // RCH, out_body, 0)

  return degk


_deg_count = _make_deg_count()


def _make_edge_gather():
  sds = jax.ShapeDtypeStruct

  @functools.partial(
      pl.kernel,
      out_type=(
          sds((EE, CW), jnp.float32),
          sds((EE, CW), jnp.float32),
      ),
      mesh=_MESH,
      scratch_types=[
          pltpu.VMEM((EK,), jnp.int32),
          pltpu.VMEM((EK,), jnp.int32),
          pltpu.VMEM((EK, CW), jnp.float32),
          pltpu.VMEM((EK, CW), jnp.float32),
          pltpu.SemaphoreType.DMA,
      ],
  )
  def eg(tab, src, dst, ocs, ocd, sidx, didx, bcs, bcd, sem):
    c = lax.axis_index("c")
    s = lax.axis_index("s")
    wid = s * NC + c
    ebase = wid * EPT

    def body(j, carry):
      off = ebase + j * EK
      pltpu.sync_copy(src.at[pl.ds(off, EK)], sidx)
      pltpu.sync_copy(dst.at[pl.ds(off, EK)], didx)
      c1 = pltpu.async_copy(tab.at[sidx], bcs, sem)
      c2 = pltpu.async_copy(tab.at[didx], bcd, sem)
      c1.wait()
      c2.wait()
      pltpu.sync_copy(bcs, ocs.at[pl.ds(off, EK)])
      pltpu.sync_copy(bcd, ocd.at[pl.ds(off, EK)])
      return carry

    lax.fori_loop(0, EPT // EK, body, 0)

  return eg


_edge_gather = _make_edge_gather()


BN = 512
BE = 512


def _prep_body(p0_ref, p1_ref, z_ref, dinv_ref, y_ref):
  deg = p0_ref[:, 0] + p1_ref[:, 0] + 1.0
  dinv = lax.rsqrt(jnp.maximum(deg, 1.0))
  dinv_ref[...] = dinv
  y_ref[...] = z_ref[...] * dinv[:, None]


def _tc_prep(p0, p1, z_pad):
  return pl.pallas_call(
      _prep_body,
      grid=(NP // BN,),
      in_specs=[
          pl.BlockSpec((BN, HH), lambda i: (i, 0)),
          pl.BlockSpec((BN, HH), lambda i: (i, 0)),
          pl.BlockSpec((BN, DD), lambda i: (i, 0)),
      ],
      out_specs=[
          pl.BlockSpec((BN,), lambda i: (i,)),
          pl.BlockSpec((BN, DD), lambda i: (i, 0)),
      ],
      out_shape=[
          jax.ShapeDtypeStruct((NP,), jnp.float32),
          jax.ShapeDtypeStruct((NP, DD), jnp.float32),
      ],
  )(p0, p1, z_pad)


def _make_layer_body(final):
  def body(p0_ref, p1_ref, y_ref, dinv_ref, w_ref, b_ref, o_ref):
    dinv = dinv_ref[...]
    m = p0_ref[...] + p1_ref[...] - y_ref[...]
    agg = m * dinv[:, None]
    h = jnp.dot(agg, w_ref[...], preferred_element_type=jnp.float32)
    h = h + b_ref[...][None, :]
    if final:
      o_ref[...] = h
    else:
      o_ref[...] = jnp.maximum(h, 0.0) * dinv[:, None]
  return body


def _tc_layer(p0, p1, y, dinv, w, b, final):
  hin, hout = w.shape
  return pl.pallas_call(
      _make_layer_body(final),
      grid=(NP // BN,),
      in_specs=[
          pl.BlockSpec((BN, hin), lambda i: (i, 0)),
          pl.BlockSpec((BN, hin), lambda i: (i, 0)),
          pl.BlockSpec((BN, hin), lambda i: (i, 0)),
          pl.BlockSpec((BN,), lambda i: (i,)),
          pl.BlockSpec((hin, hout), lambda i: (0, 0)),
          pl.BlockSpec((hout,), lambda i: (0,)),
      ],
      out_specs=pl.BlockSpec((BN, hout), lambda i: (i, 0)),
      out_shape=jax.ShapeDtypeStruct((NP, hout), jnp.float32),
  )(p0, p1, y, dinv, w, b)


def _head_body(cs_ref, cd_ref,
               wr1a_ref, wr1b_ref, br1_ref, wr2_ref, br2_ref,
               wt1a_ref, wt1b_ref, bt1_ref, wt2_ref, bt2_ref,
               o_ref):
  cs = cs_ref[...]
  cd = cd_ref[...]
  zs = cs[:, :DD]
  zd = cd[:, :DD]
  gs = cs[:, DD:DD + HH // 2]
  gd = cd[:, DD:DD + HH // 2]

  diff = zs - zd + 1e-6
  dist = -jnp.sqrt(jnp.sum(diff * diff, axis=1))

  def mlp(w1a, w1b, b1, w2, b2):
    u = jnp.dot(gs, w1a, preferred_element_type=jnp.float32)
    u = u + jnp.dot(gd, w1b, preferred_element_type=jnp.float32)
    u = u + b1[None, :]
    u = jnp.where(u >= 0.0, u, 0.2 * u)
    return jnp.sum(u * w2, axis=1) + b2[0, 0]

  r = mlp(wr1a_ref[...], wr1b_ref[...], br1_ref[...], wr2_ref[...],
          br2_ref[...])
  t = mlp(wt1a_ref[...], wt1b_ref[...], bt1_ref[...], wt2_ref[...],
          bt2_ref[...])
  o_ref[...] = jax.nn.sigmoid((dist - r) / t)


def _tc_head(cats, catd, wr1, br1, wr2, br2, wt1, bt1, wt2, bt2):
  hg = HH // 2
  wr1a, wr1b = wr1[:hg], wr1[hg:]
  wt1a, wt1b = wt1[:hg], wt1[hg:]
  wr2r = wr2.reshape(1, HH)
  wt2r = wt2.reshape(1, HH)
  br2r = jnp.broadcast_to(br2.reshape(1, 1), (1, 128))
  bt2r = jnp.broadcast_to(bt2.reshape(1, 1), (1, 128))
  full = lambda a, b: pl.BlockSpec((a, b), lambda i: (0, 0))
  return pl.pallas_call(
      _head_body,
      grid=(EE // BE,),
      in_specs=[
          pl.BlockSpec((BE, CW), lambda i: (i, 0)),
          pl.BlockSpec((BE, CW), lambda i: (i, 0)),
          full(hg, HH), full(hg, HH),
          pl.BlockSpec((HH,), lambda i: (0,)),
          full(1, HH), full(1, 128),
          full(hg, HH), full(hg, HH),
          pl.BlockSpec((HH,), lambda i: (0,)),
          full(1, HH), full(1, 128),
      ],
      out_specs=pl.BlockSpec((BE,), lambda i: (i,)),
      out_shape=jax.ShapeDtypeStruct((EE,), jnp.float32),
  )(cats, catd, wr1a, wr1b, br1, wr2r, br2r, wt1a, wt1b, bt1, wt2r, bt2r)


def kernel(z, edge_index, W0, b0, W1, b1, W2, b2, W3, b3,
           Wr1, br1, Wr2, br2, Wt1, bt1, Wt2, bt2):
  src = edge_index[0]
  dst = edge_index[1]
  z_pad = jnp.zeros((NP, DD), jnp.float32).at[:NN].set(z)

  zeros_tab = jnp.zeros((NP, HH), jnp.float32)
  ones_blk = jnp.ones((EK, HH), jnp.float32)
  degp = _deg_count(zeros_tab, ones_blk, dst)
  dinv, y = _tc_prep(degp[:NP], degp[NP:], z_pad)

  ws = [(W0, b0), (W1, b1), (W2, b2), (W3, b3)]
  for i, (w, b) in enumerate(ws):
    p = _seg_gather_add(y, src, dst)
    y = _tc_layer(p[:NP], p[NP:], y, dinv, w, b, final=(i == 3))
  g = y

  tab = jnp.concatenate(
      [z_pad, g, jnp.zeros((NP, CW - DD - HH // 2), jnp.float32)], axis=1)
  cats, catd = _edge_gather(tab, src, dst)
  return _tc_head(cats, catd, Wr1, br1, Wr2, br2, Wt1, bt1, Wt2, bt2)


# speedup vs baseline: 8.5600x; 1.5113x over previous
import functools

import jax
import jax.numpy as jnp
from jax import lax
from jax.experimental import pallas as pl
from jax.experimental.pallas import tpu as pltpu
from jax.experimental.pallas import tpu_sc as plsc

NN = 10000
NP = 10240
EE = 320000
DD = 128
HH = 128
CW = 256

NC = 2
NS = 16
NW = NC * NS
EPT = EE // NW
EK = 80
NCH = EPT // EK
RPT = NP // NS
RCH = 160

_MESH = plsc.VectorSubcoreMesh(core_axis_name="c", subcore_axis_name="s")


def _make_seg_gather_add():
  @functools.partial(
      pl.kernel,
      out_type=jax.ShapeDtypeStruct((NC * NP, HH), jnp.float32),
      mesh=_MESH,
      scratch_types=[
          pltpu.VMEM((EK,), jnp.int32),
          pltpu.VMEM((EK,), jnp.int32),
          pltpu.VMEM((EK,), jnp.int32),
          pltpu.VMEM((EK,), jnp.int32),
          pltpu.VMEM((EK,), jnp.int32),
          pltpu.VMEM((EK,), jnp.int32),
          pltpu.VMEM((EK,), jnp.int32),
          pltpu.VMEM((EK,), jnp.int32),
          pltpu.VMEM((EK, HH), jnp.float32),
          pltpu.VMEM((EK, HH), jnp.float32),
          pltpu.VMEM((RCH, HH), jnp.float32),
          pltpu.VMEM_SHARED((NP, HH), jnp.float32),
      ] + [pltpu.SemaphoreType.DMA] * 6,
  )
  def seg(table, src, dst, out,
          sx0, sx1, sx2, sx3, dx0, dx1, dx2, dx3,
          r0, r1, stage, acc,
          i0, i1, i2, i3, g0, g1):
    c = lax.axis_index("c")
    s = lax.axis_index("s")
    wid = s * NC + c
    sx = [sx0, sx1, sx2, sx3]
    dx = [dx0, dx1, dx2, dx3]
    rows = [r0, r1]
    isem = [i0, i1, i2, i3]
    gsem = [g0, g1]

    rbase = s * RPT

    def init_body(i, carry):
      off = rbase + i * RCH
      pltpu.sync_copy(table.at[pl.ds(off, RCH)], stage)
      pltpu.sync_copy(stage, acc.at[pl.ds(off, RCH)])
      return carry

    lax.fori_loop(0, RPT // RCH, init_body, 0)
    plsc.subcore_barrier()

    ebase = wid * EPT

    def iload(j, b):
      off = ebase + j * EK
      pltpu.async_copy(src.at[pl.ds(off, EK)], sx[b], isem[b])
      pltpu.async_copy(dst.at[pl.ds(off, EK)], dx[b], isem[b])

    def iwait(j, b):
      off = ebase + j * EK
      pltpu.make_async_copy(src.at[pl.ds(off, EK)], sx[b], isem[b]).wait()
      pltpu.make_async_copy(dst.at[pl.ds(off, EK)], dx[b], isem[b]).wait()

    def gather(ib, rb):
      pltpu.async_copy(table.at[sx[ib]], rows[rb], gsem[rb])

    def gwait(ib, rb):
      pltpu.make_async_copy(table.at[sx[ib]], rows[rb], gsem[rb]).wait()

    def scat(ib, rb):
      pltpu.sync_copy(rows[rb], acc.at[dx[ib]], add=True)

    iload(0, 0)
    iload(1, 1)
    iload(2, 2)
    iwait(0, 0)
    gather(0, 0)

    def outer(jj, carry):
      for b in range(4):
        j = jj * 4 + b

        @pl.when(j + 3 < NCH)
        def _():
          iload(j + 3, (b + 3) % 4)

        @pl.when(j + 1 < NCH)
        def _():
          iwait(j + 1, (b + 1) % 4)
          gather((b + 1) % 4, (b + 1) % 2)

        gwait(b % 4, b % 2)
        scat(b % 4, b % 2)
      return carry

    lax.fori_loop(0, (NCH - 1) // 4, outer, 0)
    gwait(0, 0)
    scat(0, 0)
    plsc.subcore_barrier()

    obase = c * NP + rbase

    def out_body(i, carry):
      pltpu.sync_copy(acc.at[pl.ds(rbase + i * RCH, RCH)], stage)
      pltpu.sync_copy(stage, out.at[pl.ds(obase + i * RCH, RCH)])
      return carry

    lax.fori_loop(0, RPT // RCH, out_body, 0)

  return seg


_seg_gather_add = _make_seg_gather_add()


def _make_edge_gather():
  sds = jax.ShapeDtypeStruct

  @functools.partial(
      pl.kernel,
      out_type=(
          sds((EE, CW), jnp.float32),
          sds((EE, CW), jnp.float32),
      ),
      mesh=_MESH,
      scratch_types=[
          pltpu.VMEM((EK,), jnp.int32),
          pltpu.VMEM((EK,), jnp.int32),
          pltpu.VMEM((EK,), jnp.int32),
          pltpu.VMEM((EK,), jnp.int32),
          pltpu.VMEM((EK, CW), jnp.float32),
          pltpu.VMEM((EK, CW), jnp.float32),
          pltpu.VMEM((EK, CW), jnp.float32),
          pltpu.VMEM((EK, CW), jnp.float32),
      ] + [pltpu.SemaphoreType.DMA] * 10,
  )
  def eg(tab, src, dst, ocs, ocd,
         sx0, sx1, dx0, dx1, cs0, cs1, cd0, cd1,
         i0, i1, ga0, ga1, gb0, gb1, wa0, wa1, wb0, wb1):
    c = lax.axis_index("c")
    s = lax.axis_index("s")
    wid = s * NC + c
    sx = [sx0, sx1]
    dx = [dx0, dx1]
    bcs = [cs0, cs1]
    bcd = [cd0, cd1]
    isem = [i0, i1]
    gsa = [ga0, ga1]
    gsb = [gb0, gb1]
    wsa = [wa0, wa1]
    wsb = [wb0, wb1]
    ebase = wid * EPT

    def iload(j, b):
      off = ebase + j * EK
      pltpu.async_copy(src.at[pl.ds(off, EK)], sx[b], isem[b])
      pltpu.async_copy(dst.at[pl.ds(off, EK)], dx[b], isem[b])

    def iwait(j, b):
      off = ebase + j * EK
      pltpu.make_async_copy(src.at[pl.ds(off, EK)], sx[b], isem[b]).wait()
      pltpu.make_async_copy(dst.at[pl.ds(off, EK)], dx[b], isem[b]).wait()

    def gathers(b):
      pltpu.async_copy(tab.at[sx[b]], bcs[b], gsa[b])
      pltpu.async_copy(tab.at[dx[b]], bcd[b], gsb[b])

    def gwaits(b):
      pltpu.make_async_copy(tab.at[sx[b]], bcs[b], gsa[b]).wait()
      pltpu.make_async_copy(tab.at[dx[b]], bcd[b], gsb[b]).wait()

    def writes(j, b):
      off = ebase + j * EK
      pltpu.async_copy(bcs[b], ocs.at[pl.ds(off, EK)], wsa[b])
      pltpu.async_copy(bcd[b], ocd.at[pl.ds(off, EK)], wsb[b])

    def wwaits(j, b):
      off = ebase + j * EK
      pltpu.make_async_copy(bcs[b], ocs.at[pl.ds(off, EK)], wsa[b]).wait()
      pltpu.make_async_copy(bcd[b], ocd.at[pl.ds(off, EK)], wsb[b]).wait()

    iload(0, 0)
    iload(1, 1)
    iwait(0, 0)
    gathers(0)

    def outer(jj, carry):
      for b in range(2):
        j = jj * 2 + b
        nb = 1 - b

        gwaits(b)
        writes(j, b)

        @pl.when(j + 2 < NCH)
        def _():
          iload(j + 2, b)

        @pl.when(j + 1 < NCH)
        def _():
          iwait(j + 1, nb)

          @pl.when(j >= 1)
          def _():
            wwaits(j - 1, nb)
          gathers(nb)
      return carry

    lax.fori_loop(0, NCH // 2, outer, 0)
    gwaits(0)
    writes(NCH - 1, 0)
    wwaits(NCH - 2, 1)
    wwaits(NCH - 1, 0)

  return eg


_edge_gather = _make_edge_gather()


BN = 512
BE = 512


def _prep_body(p0_ref, p1_ref, z_ref, dinv_ref, y_ref):
  deg = p0_ref[:, 0] + p1_ref[:, 0] - 1.0
  dinv = lax.rsqrt(jnp.maximum(deg, 1.0))
  dinv_ref[...] = dinv
  y_ref[...] = z_ref[...] * dinv[:, None]


def _tc_prep(p0, p1, z_pad):
  return pl.pallas_call(
      _prep_body,
      grid=(NP // BN,),
      in_specs=[
          pl.BlockSpec((BN, HH), lambda i: (i, 0)),
          pl.BlockSpec((BN, HH), lambda i: (i, 0)),
          pl.BlockSpec((BN, DD), lambda i: (i, 0)),
      ],
      out_specs=[
          pl.BlockSpec((BN,), lambda i: (i,)),
          pl.BlockSpec((BN, DD), lambda i: (i, 0)),
      ],
      out_shape=[
          jax.ShapeDtypeStruct((NP,), jnp.float32),
          jax.ShapeDtypeStruct((NP, DD), jnp.float32),
      ],
  )(p0, p1, z_pad)


def _make_layer_body(final):
  def body(p0_ref, p1_ref, y_ref, dinv_ref, w_ref, b_ref, o_ref):
    dinv = dinv_ref[...]
    m = p0_ref[...] + p1_ref[...] - y_ref[...]
    agg = m * dinv[:, None]
    h = jnp.dot(agg, w_ref[...], preferred_element_type=jnp.float32)
    h = h + b_ref[...][None, :]
    if final:
      o_ref[...] = h
    else:
      o_ref[...] = jnp.maximum(h, 0.0) * dinv[:, None]
  return body


def _tc_layer(p0, p1, y, dinv, w, b, final):
  hin, hout = w.shape
  return pl.pallas_call(
      _make_layer_body(final),
      grid=(NP // BN,),
      in_specs=[
          pl.BlockSpec((BN, hin), lambda i: (i, 0)),
          pl.BlockSpec((BN, hin), lambda i: (i, 0)),
          pl.BlockSpec((BN, hin), lambda i: (i, 0)),
          pl.BlockSpec((BN,), lambda i: (i,)),
          pl.BlockSpec((hin, hout), lambda i: (0, 0)),
          pl.BlockSpec((hout,), lambda i: (0,)),
      ],
      out_specs=pl.BlockSpec((BN, hout), lambda i: (i, 0)),
      out_shape=jax.ShapeDtypeStruct((NP, hout), jnp.float32),
  )(p0, p1, y, dinv, w, b)


def _head_body(cs_ref, cd_ref,
               wr1a_ref, wr1b_ref, br1_ref, wr2_ref, br2_ref,
               wt1a_ref, wt1b_ref, bt1_ref, wt2_ref, bt2_ref,
               o_ref):
  cs = cs_ref[...]
  cd = cd_ref[...]
  zs = cs[:, :DD]
  zd = cd[:, :DD]
  gs = cs[:, DD:DD + HH // 2]
  gd = cd[:, DD:DD + HH // 2]

  diff = zs - zd + 1e-6
  dist = -jnp.sqrt(jnp.sum(diff * diff, axis=1))

  def mlp(w1a, w1b, b1, w2, b2):
    u = jnp.dot(gs, w1a, preferred_element_type=jnp.float32)
    u = u + jnp.dot(gd, w1b, preferred_element_type=jnp.float32)
    u = u + b1[None, :]
    u = jnp.where(u >= 0.0, u, 0.2 * u)
    return jnp.sum(u * w2, axis=1) + b2[0, 0]

  r = mlp(wr1a_ref[...], wr1b_ref[...], br1_ref[...], wr2_ref[...],
          br2_ref[...])
  t = mlp(wt1a_ref[...], wt1b_ref[...], bt1_ref[...], wt2_ref[...],
          bt2_ref[...])
  o_ref[...] = jax.nn.sigmoid((dist - r) / t)


def _tc_head(cats, catd, wr1, br1, wr2, br2, wt1, bt1, wt2, bt2):
  hg = HH // 2
  wr1a, wr1b = wr1[:hg], wr1[hg:]
  wt1a, wt1b = wt1[:hg], wt1[hg:]
  wr2r = wr2.reshape(1, HH)
  wt2r = wt2.reshape(1, HH)
  br2r = jnp.broadcast_to(br2.reshape(1, 1), (1, 128))
  bt2r = jnp.broadcast_to(bt2.reshape(1, 1), (1, 128))
  full = lambda a, b: pl.BlockSpec((a, b), lambda i: (0, 0))
  return pl.pallas_call(
      _head_body,
      grid=(EE // BE,),
      in_specs=[
          pl.BlockSpec((BE, CW), lambda i: (i, 0)),
          pl.BlockSpec((BE, CW), lambda i: (i, 0)),
          full(hg, HH), full(hg, HH),
          pl.BlockSpec((HH,), lambda i: (0,)),
          full(1, HH), full(1, 128),
          full(hg, HH), full(hg, HH),
          pl.BlockSpec((HH,), lambda i: (0,)),
          full(1, HH), full(1, 128),
      ],
      out_specs=pl.BlockSpec((BE,), lambda i: (i,)),
      out_shape=jax.ShapeDtypeStruct((EE,), jnp.float32),
  )(cats, catd, wr1a, wr1b, br1, wr2r, br2r, wt1a, wt1b, bt1, wt2r, bt2r)


def kernel(z, edge_index, W0, b0, W1, b1, W2, b2, W3, b3,
           Wr1, br1, Wr2, br2, Wt1, bt1, Wt2, bt2):
  src = edge_index[0]
  dst = edge_index[1]
  z_pad = jnp.zeros((NP, DD), jnp.float32).at[:NN].set(z)

  ones_tab = jnp.ones((NP, HH), jnp.float32)
  degp = _seg_gather_add(ones_tab, src, dst)
  dinv, y = _tc_prep(degp[:NP], degp[NP:], z_pad)

  ws = [(W0, b0), (W1, b1), (W2, b2), (W3, b3)]
  for i, (w, b) in enumerate(ws):
    p = _seg_gather_add(y, src, dst)
    y = _tc_layer(p[:NP], p[NP:], y, dinv, w, b, final=(i == 3))
  g = y

  tab = jnp.concatenate(
      [z_pad, g, jnp.zeros((NP, CW - DD - HH // 2), jnp.float32)], axis=1)
  cats, catd = _edge_gather(tab, src, dst)
  return _tc_head(cats, catd, Wr1, br1, Wr2, br2, Wt1, bt1, Wt2, bt2)
